# baseline (speedup 1.0000x reference)
import jax
import jax.numpy as jnp
from jax.experimental import pallas as pl

_DEPTH = 1000
_ROWS_PER_BLOCK = 512


def _onehot_block(idx_ref, out_ref):
    idx = idx_ref[0, 0, :]
    iota = jax.lax.broadcasted_iota(jnp.int32, (_ROWS_PER_BLOCK, _DEPTH), 1)
    out_ref[...] = (idx[:, None] == iota).astype(jnp.float32)


def kernel(inputs):
    x = inputs.astype(jnp.int32)
    if x.ndim == 3:
        x = x[:, :, 0]
    b, s = x.shape
    n = b * s
    g = n // _ROWS_PER_BLOCK
    idx = x.reshape(g, 1, _ROWS_PER_BLOCK)
    out = pl.pallas_call(
        _onehot_block,
        grid=(g,),
        in_specs=[pl.BlockSpec((1, 1, _ROWS_PER_BLOCK), lambda i: (i, 0, 0))],
        out_specs=pl.BlockSpec((_ROWS_PER_BLOCK, _DEPTH), lambda i: (i, 0)),
        out_shape=jax.ShapeDtypeStruct((n, _DEPTH), jnp.float32),
    )(idx)
    return out.reshape(b, s, _DEPTH)


# speedup vs baseline: 1.0665x; 1.0665x over previous
import jax
import jax.numpy as jnp
from jax.experimental import pallas as pl

_DEPTH = 1000
_ROWS_PER_BLOCK = 2048


def _onehot_block(idx_ref, out_ref):
    idx = idx_ref[0, 0, :]
    iota = jax.lax.broadcasted_iota(jnp.int32, (_ROWS_PER_BLOCK, _DEPTH), 1)
    out_ref[...] = (idx[:, None] == iota).astype(jnp.float32)


def kernel(inputs):
    x = inputs.astype(jnp.int32)
    if x.ndim == 3:
        x = x[:, :, 0]
    b, s = x.shape
    n = b * s
    g = n // _ROWS_PER_BLOCK
    idx = x.reshape(g, 1, _ROWS_PER_BLOCK)
    out = pl.pallas_call(
        _onehot_block,
        grid=(g,),
        in_specs=[pl.BlockSpec((1, 1, _ROWS_PER_BLOCK), lambda i: (i, 0, 0))],
        out_specs=pl.BlockSpec((_ROWS_PER_BLOCK, _DEPTH), lambda i: (i, 0)),
        out_shape=jax.ShapeDtypeStruct((n, _DEPTH), jnp.float32),
    )(idx)
    return out.reshape(b, s, _DEPTH)
